# baseline (speedup 1.0000x reference)
import functools

import jax
import jax.numpy as jnp
from jax import lax
from jax.experimental import pallas as pl
from jax.experimental.pallas import tpu as pltpu
from jax.experimental.pallas import tpu_sc as plsc

VOCAB = 1000000
MAXLEN = 200
DIM = 64
BATCH = 4096
SEQ = 200

NW = 32
BPW = BATCH // NW
LT = SEQ // 8
CT = BATCH // 128
DT = DIM // 8
NBUF = 4
PADW = 65

_mesh = plsc.VectorSubcoreMesh(core_axis_name="c", subcore_axis_name="s")


@functools.partial(
    pl.kernel,
    out_type=jax.ShapeDtypeStruct((SEQ, DT, CT, 8, 128), jnp.float32),
    mesh=_mesh,
    compiler_params=pltpu.CompilerParams(
        use_tc_tiling_on_sc=False,
        needs_layout_passes=False,
        disable_bounds_checks=True,
    ),
    scratch_types=[
        pltpu.VMEM((LT, 8, 128), jnp.int32),
        pltpu.VMEM((SEQ, DIM), jnp.float32),
        pltpu.VMEM((NBUF, BPW, DIM), jnp.float32),
        pltpu.VMEM((BPW * PADW,), jnp.float32),
        pltpu.VMEM((NBUF, DT, 8, 128), jnp.float32),
        pltpu.SemaphoreType.DMA,
        pltpu.SemaphoreType.DMA,
    ],
)
def _emb_kernel(x4_hbm, tok_hbm, pos_hbm, out_hbm, idx_v, pos_v, grows_v, pad_v, otile_v, gsem, ssem):
    c = lax.axis_index("s") * 2 + lax.axis_index("c")
    pltpu.sync_copy(x4_hbm.at[:, c], idx_v)
    pltpu.sync_copy(pos_hbm, pos_v)

    def gather_l(l, s):
        return pltpu.async_copy(
            tok_hbm.at[idx_v.at[l // 8, l % 8]], grows_v.at[s], gsem
        )

    for s in range(NBUF):
        gather_l(s, s)

    iota16 = lax.iota(jnp.int32, 16)
    rowbase = [(iota16 + 16 * bg) * PADW for bg in range(8)]

    def transpose_add(l, s):
        pvs = [pos_v[l, pl.ds(16 * g, 16)] for g in range(4)]

        def b_body(b, carry):
            base = b * PADW
            for g in range(4):
                pad_v[pl.ds(base + 16 * g, 16)] = (
                    grows_v[s, b, pl.ds(16 * g, 16)] + pvs[g]
                )
            return carry

        lax.fori_loop(0, BPW, b_body, 0, unroll=8)

        def d_body(d, carry):
            dsp = jnp.full((16,), d, jnp.int32)
            for bg in range(8):
                v = plsc.load_gather(pad_v, [rowbase[bg] + dsp])
                otile_v[s, d // 8, d % 8, pl.ds(16 * bg, 16)] = v
            return carry

        lax.fori_loop(0, DIM, d_body, 0, unroll=8)

    def outer(i, carry):
        for s in range(NBUF):
            l = i * NBUF + s

            @pl.when(i > 0)
            def _wait_out_slot():
                pltpu.make_async_copy(otile_v.at[s], out_hbm.at[0, :, 0], ssem).wait()

            pltpu.make_async_copy(
                tok_hbm.at[idx_v.at[0, 0]], grows_v.at[s], gsem
            ).wait()
            transpose_add(l, s)

            @pl.when(l + NBUF < SEQ)
            def _next_gather():
                gather_l(l + NBUF, s)

            pltpu.async_copy(otile_v.at[s], out_hbm.at[l, :, c], ssem)
        return carry

    lax.fori_loop(0, SEQ // NBUF, outer, 0)
    for s in range(NBUF):
        pltpu.make_async_copy(otile_v.at[s], out_hbm.at[0, :, 0], ssem).wait()


def kernel(x, tok_table, pos_table):
    x4 = x.T.reshape(LT, 8, CT, 128).transpose(0, 2, 1, 3)
    out5 = _emb_kernel(x4, tok_table, pos_table)
    return out5.transpose(2, 4, 0, 1, 3).reshape(BATCH, SEQ, DIM)


# speedup vs baseline: 1.9885x; 1.9885x over previous
import functools

import jax
import jax.numpy as jnp
from jax import lax
from jax.experimental import pallas as pl
from jax.experimental.pallas import tpu as pltpu
from jax.experimental.pallas import tpu_sc as plsc

VOCAB = 1000000
MAXLEN = 200
DIM = 64
BATCH = 4096
SEQ = 200

NW = 32
BPW = BATCH // NW
LT = SEQ // 8
CT = BATCH // 128
DT = DIM // 8
NBUF = 4
PADW = 65

_mesh = plsc.VectorSubcoreMesh(core_axis_name="c", subcore_axis_name="s")


@functools.partial(
    pl.kernel,
    out_type=jax.ShapeDtypeStruct((SEQ, DT, CT, 8, 128), jnp.float32),
    mesh=_mesh,
    compiler_params=pltpu.CompilerParams(
        use_tc_tiling_on_sc=False,
        needs_layout_passes=False,
        disable_bounds_checks=True,
    ),
    scratch_types=[
        pltpu.VMEM((LT, 8, 128), jnp.int32),
        pltpu.VMEM((SEQ, DIM), jnp.float32),
        pltpu.VMEM((NBUF, BPW, DIM), jnp.float32),
        pltpu.VMEM((BPW * PADW,), jnp.float32),
        pltpu.VMEM((NBUF, DT, 8, 128), jnp.float32),
        pltpu.SemaphoreType.DMA,
        pltpu.SemaphoreType.DMA,
    ],
)
def _emb_kernel(x4_hbm, tok_hbm, pos_hbm, out_hbm, idx_v, pos_v, grows_v, pad_v, otile_v, gsem, ssem):
    c = lax.axis_index("s") * 2 + lax.axis_index("c")
    pltpu.sync_copy(x4_hbm.at[:, c], idx_v)
    pltpu.sync_copy(pos_hbm, pos_v)

    def gather_l(l, s):
        return pltpu.async_copy(
            tok_hbm.at[idx_v.at[l // 8, l % 8]], grows_v.at[s], gsem
        )

    for s in range(NBUF):
        gather_l(s, s)

    iota16 = lax.iota(jnp.int32, 16)
    rowbase = [(iota16 + 16 * bg) * PADW for bg in range(8)]

    def transpose_add(l, s):
        pvs = [pos_v[l, pl.ds(16 * g, 16)] for g in range(4)]

        @plsc.parallel_loop(0, BPW, 1, unroll=8)
        def _pass1(b):
            base = b * PADW
            for g in range(4):
                pad_v[pl.ds(base + 16 * g, 16)] = (
                    grows_v[s, b, pl.ds(16 * g, 16)] + pvs[g]
                )

        @plsc.parallel_loop(0, DIM, 1, unroll=8)
        def _pass2(d):
            dsp = jnp.full((16,), d, jnp.int32)
            for bg in range(8):
                v = plsc.load_gather(pad_v, [rowbase[bg] + dsp])
                otile_v[s, d // 8, d % 8, pl.ds(16 * bg, 16)] = v

    def outer(i, carry):
        for s in range(NBUF):
            l = i * NBUF + s

            @pl.when(i > 0)
            def _wait_out_slot():
                pltpu.make_async_copy(otile_v.at[s], out_hbm.at[0, :, 0], ssem).wait()

            pltpu.make_async_copy(
                tok_hbm.at[idx_v.at[0, 0]], grows_v.at[s], gsem
            ).wait()
            transpose_add(l, s)

            @pl.when(l + NBUF < SEQ)
            def _next_gather():
                gather_l(l + NBUF, s)

            pltpu.async_copy(otile_v.at[s], out_hbm.at[l, :, c], ssem)
        return carry

    lax.fori_loop(0, SEQ // NBUF, outer, 0)
    for s in range(NBUF):
        pltpu.make_async_copy(otile_v.at[s], out_hbm.at[0, :, 0], ssem).wait()


def kernel(x, tok_table, pos_table):
    x4 = x.T.reshape(LT, 8, CT, 128).transpose(0, 2, 1, 3)
    out5 = _emb_kernel(x4, tok_table, pos_table)
    return out5.transpose(2, 4, 0, 1, 3).reshape(BATCH, SEQ, DIM)
